# Initial kernel scaffold; baseline (speedup 1.0000x reference)
import functools

import jax
import jax.numpy as jnp
from jax import lax
from jax.experimental import pallas as pl
from jax.experimental.pallas import tpu as pltpu
from jax.experimental.pallas import tpu_sc as plsc

B, S, W, H = 4, 2500, 128, 128
PIX = W * H
N = B * S
E = 160000
C = 256
AUX = 16
FW = AUX + 3 * C
NC, NS = 2, 16
EPAD = 163840
EW = 128
PW = 64
TROWS = 157
NPAD = N + 16
DROWS = NPAD // NS
RB = 1000
GRID = N // RB
EPS = 1e-5

_sc_mesh = plsc.VectorSubcoreMesh(core_axis_name="c", subcore_axis_name="s")


@functools.partial(
    pl.kernel,
    out_type=jax.ShapeDtypeStruct((B, S, FW), jnp.float32),
    mesh=_sc_mesh,
    scratch_types=[
        pltpu.VMEM((PW, FW), jnp.float32),
        pltpu.VMEM((PW,), jnp.int32),
        pltpu.VMEM_SHARED((S, FW), jnp.float32),
        pltpu.SemaphoreType.DMA,
    ],
)
def _pool_kernel(feat, lbl, out, buf, idx, table, sem):
    c = lax.axis_index("c")
    t = lax.axis_index("s")
    zf = jnp.zeros((16,), jnp.float32)

    def _zero_buf():
        def zrow(r, carry):
            for j in range(FW // 16):
                buf[r, pl.ds(16 * j, 16)] = zf
            return carry
        lax.fori_loop(0, PW, zrow, 0)

    start = jnp.where(t == NS - 1, S - TROWS, t * TROWS)

    def _zero_table():
        pltpu.sync_copy(buf, table.at[pl.ds(start, PW)])
        pltpu.sync_copy(buf, table.at[pl.ds(start + PW, PW)])
        pltpu.sync_copy(buf.at[pl.ds(0, TROWS - 2 * PW)],
                        table.at[pl.ds(start + 2 * PW, TROWS - 2 * PW)])

    _zero_buf()
    _zero_table()
    plsc.subcore_barrier()
    for img_i in range(B // NC):
        img = c * (B // NC) + img_i
        base = t * (PIX // NS)
        for w in range(PIX // NS // PW):
            ws = base + w * PW
            pltpu.sync_copy(feat.at[img, pl.ds(ws, PW)], buf)
            pltpu.sync_copy(lbl.at[img, pl.ds(ws, PW)], idx)
            pltpu.async_copy(buf, table.at[idx], sem, add=True).wait()
        plsc.subcore_barrier()
        pltpu.sync_copy(table.at[pl.ds(start, TROWS)],
                        out.at[img, pl.ds(start, TROWS)])
        if img_i + 1 < B // NC:
            plsc.subcore_barrier()
            _zero_buf()
            _zero_table()
            plsc.subcore_barrier()


@functools.partial(
    pl.kernel,
    out_type=jax.ShapeDtypeStruct((NC, NPAD, 16), jnp.float32),
    mesh=_sc_mesh,
    scratch_types=[
        pltpu.VMEM((EW, 16), jnp.float32),
        pltpu.VMEM((EW, 16), jnp.float32),
        pltpu.VMEM((EW,), jnp.int32),
        pltpu.VMEM((EW,), jnp.int32),
        pltpu.VMEM_SHARED((NPAD, 16), jnp.float32),
        pltpu.SemaphoreType.DMA,
    ],
)
def _deg_kernel(dst_h, lab_h, out, upd, zbuf, ibuf, lbuf, table, sem):
    c = lax.axis_index("c")
    t = lax.axis_index("s")
    zf = jnp.zeros((16,), jnp.float32)
    one0 = jnp.where(lax.iota(jnp.int32, 16) == 0, 1.0, 0.0)

    def zrow(r, carry):
        upd[r, :] = one0
        zbuf[r, :] = zf
        return carry
    lax.fori_loop(0, EW, zrow, 0)

    base_r = t * DROWS
    for k in range(4):
        pltpu.sync_copy(zbuf, table.at[pl.ds(base_r + EW * k, EW)])
    pltpu.sync_copy(zbuf.at[pl.ds(0, DROWS - 4 * EW)],
                    table.at[pl.ds(base_r + 4 * EW, DROWS - 4 * EW)])
    plsc.subcore_barrier()

    ept = EPAD // (NC * NS)
    ebase = (c * NS + t) * ept

    def win(w, carry):
        es = ebase + w * EW
        pltpu.sync_copy(dst_h.at[pl.ds(es, EW)], ibuf)
        pltpu.sync_copy(lab_h.at[pl.ds(es, EW)], lbuf)

        def cw(j, carry2):
            sl = pl.ds(16 * j, 16)
            d = ibuf[sl]
            l = lbuf[sl]
            ibuf[sl] = jnp.where(l != -1, d, N + t)
            return carry2
        lax.fori_loop(0, EW // 16, cw, 0)
        pltpu.async_copy(upd, table.at[ibuf], sem, add=True).wait()
        return carry
    lax.fori_loop(0, ept // EW, win, 0)
    plsc.subcore_barrier()
    pltpu.sync_copy(table.at[pl.ds(base_r, DROWS)],
                    out.at[c, pl.ds(base_r, DROWS)])


@functools.partial(
    pl.kernel,
    out_type=jax.ShapeDtypeStruct((NC, NPAD, C // 2), jnp.float32),
    mesh=_sc_mesh,
    scratch_types=[
        pltpu.VMEM((EW, C // 2), jnp.float32),
        pltpu.VMEM((EW, C // 2), jnp.float32),
        pltpu.VMEM((EW,), jnp.int32),
        pltpu.VMEM((EW,), jnp.int32),
        pltpu.VMEM((EW,), jnp.int32),
        pltpu.VMEM_SHARED((NPAD, C // 2), jnp.float32),
        pltpu.SemaphoreType.DMA,
        pltpu.SemaphoreType.DMA,
    ],
)
def _agg_kernel(z, src_h, dst_h, lab_h, out, rows, zbuf, sbuf, tbuf, lbuf,
                table, sem, sem2):
    c = lax.axis_index("c")
    t = lax.axis_index("s")
    zf = jnp.zeros((16,), jnp.float32)

    def zrow(r, carry):
        for j in range(C // 2 // 16):
            zbuf[r, pl.ds(16 * j, 16)] = zf
        return carry
    lax.fori_loop(0, EW, zrow, 0)

    base_r = t * DROWS
    for k in range(4):
        pltpu.sync_copy(zbuf, table.at[pl.ds(base_r + EW * k, EW)])
    pltpu.sync_copy(zbuf.at[pl.ds(0, DROWS - 4 * EW)],
                    table.at[pl.ds(base_r + 4 * EW, DROWS - 4 * EW)])
    plsc.subcore_barrier()

    ept = EPAD // NS
    ebase = t * ept

    def win(w, carry):
        es = ebase + w * EW
        pltpu.sync_copy(src_h.at[pl.ds(es, EW)], sbuf)
        pltpu.sync_copy(dst_h.at[pl.ds(es, EW)], tbuf)
        pltpu.sync_copy(lab_h.at[pl.ds(es, EW)], lbuf)

        def cw(j, carry2):
            sl = pl.ds(16 * j, 16)
            d = tbuf[sl]
            l = lbuf[sl]
            tbuf[sl] = jnp.where(l != -1, d, N + t)
            sbuf[sl] = sbuf[sl] + c * N
            return carry2
        lax.fori_loop(0, EW // 16, cw, 0)
        pltpu.async_copy(z.at[sbuf], rows, sem).wait()
        pltpu.async_copy(rows, table.at[tbuf], sem2, add=True).wait()
        return carry
    lax.fori_loop(0, ept // EW, win, 0)
    plsc.subcore_barrier()
    pltpu.sync_copy(table.at[pl.ds(base_r, DROWS)],
                    out.at[c, pl.ds(base_r, DROWS)])


def _row_spec(cols):
    return pl.BlockSpec((RB, cols), lambda i: (i, 0))


def _full_spec(shape):
    nd = len(shape)
    return pl.BlockSpec(shape, lambda i: (0,) * nd)


def _z_spec():
    return pl.BlockSpec((NC, RB, C // 2), lambda i: (0, i, 0))


def _p1_body(psum_ref, degp_ref, gw_ref, coords_ref, dinv_ref, z_ref,
             cstats_ref, acc):
    i = pl.program_id(0)
    p = psum_ref[...]
    cnt = p[:, 4]
    inv = 1.0 / jnp.maximum(cnt, 1.0)
    x0 = p[:, AUX:AUX + C] * inv[:, None]
    xw = jnp.dot(x0, gw_ref[...], preferred_element_type=jnp.float32)
    degp = degp_ref[...]
    deg = degp[0, :, 0] + degp[1, :, 0] + 1.0
    dinv = lax.rsqrt(deg)
    z = xw * dinv[:, None]
    z_ref[0] = z[:, :C // 2]
    z_ref[1] = z[:, C // 2:]
    craw = p[:, 0:4] * inv[:, None]
    coords_ref[...] = craw

    @pl.when(i == 0)
    def _():
        acc[...] = jnp.zeros_like(acc)

    acc[0, :] += jnp.sum(craw, axis=0)
    acc[1, :] += jnp.sum(craw * craw, axis=0)
    cstats_ref[...] = acc[...]
    dinv_ref[...] = dinv[:, None]


def _p1(psum, degp, gw0):
    return pl.pallas_call(
        _p1_body,
        grid=(GRID,),
        in_specs=[
            _row_spec(FW),
            pl.BlockSpec((NC, RB, 16), lambda i: (0, i, 0)),
            _full_spec((C, C)),
        ],
        out_specs=[
            _row_spec(4),
            _row_spec(1),
            _z_spec(),
            _full_spec((2, 4)),
        ],
        out_shape=[
            jax.ShapeDtypeStruct((N, 4), jnp.float32),
            jax.ShapeDtypeStruct((N, 1), jnp.float32),
            jax.ShapeDtypeStruct((NC, N, C // 2), jnp.float32),
            jax.ShapeDtypeStruct((2, 4), jnp.float32),
        ],
        scratch_shapes=[pltpu.VMEM((2, 4), jnp.float32)],
    )(psum, degp, gw0)


def _pa_body(agg_ref, z_ref, dinv_ref, gb_ref, t_ref, stats_ref, acc):
    i = pl.program_id(0)
    a = agg_ref[...]
    zz = z_ref[...]
    tl = jnp.concatenate([a[0] + zz[0], a[1] + zz[1]], axis=1)
    t = tl * dinv_ref[...] + gb_ref[0][None, :]
    t_ref[...] = t

    @pl.when(i == 0)
    def _():
        acc[...] = jnp.zeros_like(acc)

    acc[0, :] += jnp.sum(t, axis=0)
    acc[1, :] += jnp.sum(t * t, axis=0)
    stats_ref[...] = acc[...]


def _pa(agg, z, dinv, gb):
    return pl.pallas_call(
        _pa_body,
        grid=(GRID,),
        in_specs=[
            pl.BlockSpec((NC, RB, C // 2), lambda i: (0, i, 0)),
            _z_spec(),
            _row_spec(1),
            _full_spec((1, C)),
        ],
        out_specs=[_row_spec(C), _full_spec((2, C))],
        out_shape=[
            jax.ShapeDtypeStruct((N, C), jnp.float32),
            jax.ShapeDtypeStruct((2, C), jnp.float32),
        ],
        scratch_shapes=[pltpu.VMEM((2, C), jnp.float32)],
    )(agg, z, dinv, gb)


def _bn(x, stats, g, b):
    mu = stats[0] / N
    var = stats[1] / N - mu * mu
    return (x - mu[None, :]) * lax.rsqrt(var + EPS)[None, :] * g + b


def _pb_body(t_ref, stats_ref, g_ref, b_ref, psum_ref, mw_ref, mb_ref,
             u_ref, ustats_ref, acc, *, col):
    i = pl.program_id(0)
    h = jnp.maximum(_bn(t_ref[...], stats_ref[...], g_ref[0][None, :],
                        b_ref[0][None, :]), 0.0)
    p = psum_ref[...]
    inv = 1.0 / jnp.maximum(p[:, 4], 1.0)
    pool = p[:, col:col + C] * inv[:, None]
    x1 = jnp.concatenate([h, pool], axis=1)
    u = jnp.dot(x1, mw_ref[...], preferred_element_type=jnp.float32) \
        + mb_ref[0][None, :]
    u_ref[...] = u

    @pl.when(i == 0)
    def _():
        acc[...] = jnp.zeros_like(acc)

    acc[0, :] += jnp.sum(u, axis=0)
    acc[1, :] += jnp.sum(u * u, axis=0)
    ustats_ref[...] = acc[...]


def _pb(t, stats, g, b, psum, mw, mb, col):
    return pl.pallas_call(
        functools.partial(_pb_body, col=col),
        grid=(GRID,),
        in_specs=[
            _row_spec(C),
            _full_spec((2, C)),
            _full_spec((1, C)),
            _full_spec((1, C)),
            _row_spec(FW),
            _full_spec((2 * C, C)),
            _full_spec((1, C)),
        ],
        out_specs=[_row_spec(C), _full_spec((2, C))],
        out_shape=[
            jax.ShapeDtypeStruct((N, C), jnp.float32),
            jax.ShapeDtypeStruct((2, C), jnp.float32),
        ],
        scratch_shapes=[pltpu.VMEM((2, C), jnp.float32)],
    )(t, stats, g, b, psum, mw, mb)


def _pc_body(u_ref, stats_ref, g_ref, b_ref, gw_ref, dinv_ref, z_ref):
    y = _bn(u_ref[...], stats_ref[...], g_ref[0][None, :], b_ref[0][None, :])
    z = jnp.dot(y, gw_ref[...], preferred_element_type=jnp.float32) \
        * dinv_ref[...]
    z_ref[0] = z[:, :C // 2]
    z_ref[1] = z[:, C // 2:]


def _pc(u, stats, g, b, gw, dinv):
    return pl.pallas_call(
        _pc_body,
        grid=(GRID,),
        in_specs=[
            _row_spec(C),
            _full_spec((2, C)),
            _full_spec((1, C)),
            _full_spec((1, C)),
            _full_spec((C, C)),
            _row_spec(1),
        ],
        out_specs=_z_spec(),
        out_shape=jax.ShapeDtypeStruct((NC, N, C // 2), jnp.float32),
    )(u, stats, g, b, gw, dinv)


def _pd_body(t_ref, stats_ref, g_ref, b_ref, craw_ref, cstats_ref, pg_ref,
             pb_ref, mw_ref, mb_ref, u_ref, ustats_ref, acc):
    i = pl.program_id(0)
    h = jnp.maximum(_bn(t_ref[...], stats_ref[...], g_ref[0][None, :],
                        b_ref[0][None, :]), 0.0)
    cmu = cstats_ref[0] / N
    cvar = cstats_ref[1] / N - cmu * cmu
    cb = jnp.maximum((craw_ref[...] - cmu[None, :]) * lax.rsqrt(cvar + EPS)
                     * pg_ref[0][None, :] + pb_ref[0][None, :], 0.0)
    xc = jnp.concatenate([h, cb], axis=1)
    u = jnp.dot(xc, mw_ref[...], preferred_element_type=jnp.float32) \
        + mb_ref[0][None, :]
    u_ref[...] = u

    @pl.when(i == 0)
    def _():
        acc[...] = jnp.zeros_like(acc)

    acc[0, :] += jnp.sum(u, axis=0)
    acc[1, :] += jnp.sum(u * u, axis=0)
    ustats_ref[...] = acc[...]


def _pd(t, stats, g, b, craw, cstats, pg, pbb, mw, mb):
    return pl.pallas_call(
        _pd_body,
        grid=(GRID,),
        in_specs=[
            _row_spec(C),
            _full_spec((2, C)),
            _full_spec((1, C)),
            _full_spec((1, C)),
            _row_spec(4),
            _full_spec((2, 4)),
            _full_spec((1, 4)),
            _full_spec((1, 4)),
            _full_spec((C + 4, C)),
            _full_spec((1, C)),
        ],
        out_specs=[_row_spec(C), _full_spec((2, C))],
        out_shape=[
            jax.ShapeDtypeStruct((N, C), jnp.float32),
            jax.ShapeDtypeStruct((2, C), jnp.float32),
        ],
        scratch_shapes=[pltpu.VMEM((2, C), jnp.float32)],
    )(t, stats, g, b, craw, cstats, pg, pbb, mw, mb)


def _pe_body(u_ref, stats_ref, g_ref, b_ref, lw_ref, lb_ref, o_ref):
    y = _bn(u_ref[...], stats_ref[...], g_ref[0][None, :], b_ref[0][None, :])
    o_ref[...] = jnp.maximum(
        jnp.dot(y, lw_ref[...], preferred_element_type=jnp.float32)
        + lb_ref[0][None, :], 0.0)


def _pe(u, stats, g, b, lw, lb):
    return pl.pallas_call(
        _pe_body,
        grid=(GRID,),
        in_specs=[
            _row_spec(C),
            _full_spec((2, C)),
            _full_spec((1, C)),
            _full_spec((1, C)),
            _full_spec((C, 128)),
            _full_spec((1, 128)),
        ],
        out_specs=_row_spec(128),
        out_shape=jax.ShapeDtypeStruct((N, 128), jnp.float32),
    )(u, stats, g, b, lw, lb)


def kernel(fx, fy, skip0, skip1, skip2, params, labels, edges_nn):
    f32 = jnp.float32
    xx = jnp.broadcast_to(
        (jnp.arange(W, dtype=f32) / (W - 1))[:, None], (W, H)).reshape(-1)
    yy = jnp.broadcast_to(
        (jnp.arange(H, dtype=f32) / (H - 1))[None, :], (W, H)).reshape(-1)
    aux = jnp.concatenate([
        jnp.broadcast_to(jnp.stack([xx, yy], axis=-1)[None], (B, PIX, 2)),
        fx.reshape(B, PIX, 1).astype(f32),
        fy.reshape(B, PIX, 1).astype(f32),
        jnp.ones((B, PIX, 1), f32),
        jnp.zeros((B, PIX, AUX - 5), f32),
    ], axis=2)
    feat = jnp.concatenate(
        [aux] + [sk.reshape(B, C, PIX).transpose(0, 2, 1).astype(f32)
                 for sk in (skip0, skip1, skip2)], axis=2)
    lbl = labels.reshape(B, PIX).astype(jnp.int32)

    src = jnp.pad(edges_nn[0].astype(jnp.int32), (0, EPAD - E))
    dst = jnp.pad(edges_nn[1].astype(jnp.int32), (0, EPAD - E))
    lab = jnp.pad(edges_nn[2].astype(jnp.int32), (0, EPAD - E),
                  constant_values=-1)

    p = params
    row = lambda a: a.reshape(1, -1).astype(f32)

    psum = _pool_kernel(feat, lbl).reshape(N, FW)
    degp = _deg_kernel(dst, lab)
    craw, dinv, z0, cstats = _p1(psum, degp, p['gW0'].astype(f32))

    agg0 = _agg_kernel(z0.reshape(NC * N, C // 2), src, dst, lab)
    t0, st0 = _pa(agg0, z0, dinv, row(p['gb0']))
    u1, su1 = _pb(t0, st0, row(p['gbn_g0']), row(p['gbn_b0']), psum,
                  p['mW0'].astype(f32), row(p['mb0']), AUX + C)
    z1 = _pc(u1, su1, row(p['mbn_g0']), row(p['mbn_b0']),
             p['gW1'].astype(f32), dinv)

    agg1 = _agg_kernel(z1.reshape(NC * N, C // 2), src, dst, lab)
    t1, st1 = _pa(agg1, z1, dinv, row(p['gb1']))
    u2, su2 = _pb(t1, st1, row(p['gbn_g1']), row(p['gbn_b1']), psum,
                  p['mW1'].astype(f32), row(p['mb1']), AUX + 2 * C)
    z2 = _pc(u2, su2, row(p['mbn_g1']), row(p['mbn_b1']),
             p['gW2'].astype(f32), dinv)

    agg2 = _agg_kernel(z2.reshape(NC * N, C // 2), src, dst, lab)
    t2, st2 = _pa(agg2, z2, dinv, row(p['gb2']))
    u3, su3 = _pd(t2, st2, row(p['gbn_g2']), row(p['gbn_b2']), craw, cstats,
                  row(p['pre_g']), row(p['pre_b']),
                  p['mW2'].astype(f32), row(p['mb2']))
    return _pe(u3, su3, row(p['mbn_g2']), row(p['mbn_b2']),
               p['lW'].astype(f32), row(p['lb']))


# speedup vs baseline: 2.8321x; 2.8321x over previous
import functools

import jax
import jax.numpy as jnp
from jax import lax
from jax.experimental import pallas as pl
from jax.experimental.pallas import tpu as pltpu
from jax.experimental.pallas import tpu_sc as plsc

B, S, W, H = 4, 2500, 128, 128
PIX = W * H
N = B * S
E = 160000
C = 256
AUX = 16
FW = 800
FWH = FW // 2
NC, NS = 2, 16
EPAD = 163840
EW = 128
PW = 64
SPAD = 2560
TROWS = 160
NPAD = N + 112
DROWS = NPAD // NS
RB = 1000
GRID = N // RB
EPS = 1e-5

_sc_mesh = plsc.VectorSubcoreMesh(core_axis_name="c", subcore_axis_name="s")


@functools.partial(
    pl.kernel,
    out_type=jax.ShapeDtypeStruct((NC, B, SPAD, FWH), jnp.float32),
    mesh=_sc_mesh,
    compiler_params=pltpu.CompilerParams(use_tc_tiling_on_sc=False),
    scratch_types=[
        pltpu.VMEM((PW, FWH), jnp.float32),
        pltpu.VMEM((PW,), jnp.int32),
        pltpu.VMEM_SHARED((SPAD, FWH), jnp.float32),
        pltpu.SemaphoreType.DMA,
    ],
)
def _pool_kernel(feat, lbl, out, buf, idx, table, sem):
    c = lax.axis_index("c")
    t = lax.axis_index("s")
    zf = jnp.zeros((16,), jnp.float32)

    def _zero_buf():
        def zrow(r, carry):
            for j in range(FWH // 16):
                buf[r, pl.ds(16 * j, 16)] = zf
            return carry
        lax.fori_loop(0, PW, zrow, 0)

    start = t * TROWS

    def _zero_table():
        pltpu.sync_copy(buf, table.at[pl.ds(start, PW)])
        pltpu.sync_copy(buf, table.at[pl.ds(start + PW, PW)])
        pltpu.sync_copy(buf.at[pl.ds(0, TROWS - 2 * PW)],
                        table.at[pl.ds(start + 2 * PW, TROWS - 2 * PW)])

    _zero_buf()
    _zero_table()
    plsc.subcore_barrier()
    for img in range(B):
        base = t * (PIX // NS)
        for w in range(PIX // NS // PW):
            ws = base + w * PW
            pltpu.sync_copy(feat.at[c, img, pl.ds(ws, PW)], buf)
            pltpu.sync_copy(lbl.at[img, pl.ds(ws, PW)], idx)
            pltpu.async_copy(buf, table.at[idx], sem, add=True).wait()
        plsc.subcore_barrier()
        pltpu.sync_copy(table.at[pl.ds(start, TROWS)],
                        out.at[c, img, pl.ds(start, TROWS)])
        if img + 1 < B:
            plsc.subcore_barrier()
            _zero_buf()
            _zero_table()
            plsc.subcore_barrier()


@functools.partial(
    pl.kernel,
    out_type=jax.ShapeDtypeStruct((NC, NPAD, 16), jnp.float32),
    mesh=_sc_mesh,
    compiler_params=pltpu.CompilerParams(use_tc_tiling_on_sc=False),
    scratch_types=[
        pltpu.VMEM((EW, 16), jnp.float32),
        pltpu.VMEM((EW, 16), jnp.float32),
        pltpu.VMEM((EW,), jnp.int32),
        pltpu.VMEM((EW,), jnp.int32),
        pltpu.VMEM_SHARED((NPAD, 16), jnp.float32),
        pltpu.SemaphoreType.DMA,
    ],
)
def _deg_kernel(dst_h, lab_h, out, upd, zbuf, ibuf, lbuf, table, sem):
    c = lax.axis_index("c")
    t = lax.axis_index("s")
    zf = jnp.zeros((16,), jnp.float32)
    one0 = jnp.where(lax.iota(jnp.int32, 16) == 0, 1.0, 0.0)

    def zrow(r, carry):
        upd[r, :] = one0
        zbuf[r, :] = zf
        return carry
    lax.fori_loop(0, EW, zrow, 0)

    base_r = t * DROWS
    for k in range(4):
        pltpu.sync_copy(zbuf, table.at[pl.ds(base_r + EW * k, EW)])
    pltpu.sync_copy(zbuf.at[pl.ds(0, DROWS - 4 * EW)],
                    table.at[pl.ds(base_r + 4 * EW, DROWS - 4 * EW)])
    plsc.subcore_barrier()

    ept = EPAD // (NC * NS)
    ebase = (c * NS + t) * ept

    def win(w, carry):
        es = ebase + w * EW
        pltpu.sync_copy(dst_h.at[pl.ds(es, EW)], ibuf)
        pltpu.sync_copy(lab_h.at[pl.ds(es, EW)], lbuf)

        def cw(j, carry2):
            sl = pl.ds(16 * j, 16)
            d = ibuf[sl]
            l = lbuf[sl]
            ibuf[sl] = jnp.where(l != -1, d, N + t)
            return carry2
        lax.fori_loop(0, EW // 16, cw, 0)
        pltpu.async_copy(upd, table.at[ibuf], sem, add=True).wait()
        return carry
    lax.fori_loop(0, ept // EW, win, 0)
    plsc.subcore_barrier()
    pltpu.sync_copy(table.at[pl.ds(base_r, DROWS)],
                    out.at[c, pl.ds(base_r, DROWS)])


@functools.partial(
    pl.kernel,
    out_type=jax.ShapeDtypeStruct((NC, NPAD, C // 2), jnp.float32),
    mesh=_sc_mesh,
    compiler_params=pltpu.CompilerParams(use_tc_tiling_on_sc=False),
    scratch_types=[
        pltpu.VMEM((EW, C // 2), jnp.float32),
        pltpu.VMEM((EW, C // 2), jnp.float32),
        pltpu.VMEM((EW,), jnp.int32),
        pltpu.VMEM((EW,), jnp.int32),
        pltpu.VMEM((EW,), jnp.int32),
        pltpu.VMEM_SHARED((NPAD, C // 2), jnp.float32),
        pltpu.SemaphoreType.DMA,
        pltpu.SemaphoreType.DMA,
    ],
)
def _agg_kernel(z, src_h, dst_h, lab_h, out, rows, zbuf, sbuf, tbuf, lbuf,
                table, sem, sem2):
    c = lax.axis_index("c")
    t = lax.axis_index("s")
    zf = jnp.zeros((16,), jnp.float32)

    def zrow(r, carry):
        for j in range(C // 2 // 16):
            zbuf[r, pl.ds(16 * j, 16)] = zf
        return carry
    lax.fori_loop(0, EW, zrow, 0)

    base_r = t * DROWS
    for k in range(4):
        pltpu.sync_copy(zbuf, table.at[pl.ds(base_r + EW * k, EW)])
    pltpu.sync_copy(zbuf.at[pl.ds(0, DROWS - 4 * EW)],
                    table.at[pl.ds(base_r + 4 * EW, DROWS - 4 * EW)])
    plsc.subcore_barrier()

    ept = EPAD // NS
    ebase = t * ept

    def win(w, carry):
        es = ebase + w * EW
        pltpu.sync_copy(src_h.at[pl.ds(es, EW)], sbuf)
        pltpu.sync_copy(dst_h.at[pl.ds(es, EW)], tbuf)
        pltpu.sync_copy(lab_h.at[pl.ds(es, EW)], lbuf)

        def cw(j, carry2):
            sl = pl.ds(16 * j, 16)
            d = tbuf[sl]
            l = lbuf[sl]
            tbuf[sl] = jnp.where(l != -1, d, N + t)
            sbuf[sl] = sbuf[sl] + c * N
            return carry2
        lax.fori_loop(0, EW // 16, cw, 0)
        pltpu.async_copy(z.at[sbuf], rows, sem).wait()
        pltpu.async_copy(rows, table.at[tbuf], sem2, add=True).wait()
        return carry
    lax.fori_loop(0, ept // EW, win, 0)
    plsc.subcore_barrier()
    pltpu.sync_copy(table.at[pl.ds(base_r, DROWS)],
                    out.at[c, pl.ds(base_r, DROWS)])


def _row_spec(cols):
    return pl.BlockSpec((RB, cols), lambda i: (i, 0))


def _full_spec(shape):
    nd = len(shape)
    return pl.BlockSpec(shape, lambda i: (0,) * nd)


def _z_spec():
    return pl.BlockSpec((NC, RB, C // 2), lambda i: (0, i, 0))


def _p1_body(psum_ref, degp_ref, gw_ref, coords_ref, dinv_ref, z_ref,
             cstats_ref, acc):
    i = pl.program_id(0)
    p = psum_ref[...]
    cnt = p[:, 4]
    inv = 1.0 / jnp.maximum(cnt, 1.0)
    x0 = p[:, AUX:AUX + C] * inv[:, None]
    xw = jnp.dot(x0, gw_ref[...], preferred_element_type=jnp.float32)
    degp = degp_ref[...]
    deg = degp[0, :, 0] + degp[1, :, 0] + 1.0
    dinv = lax.rsqrt(deg)
    z = xw * dinv[:, None]
    z_ref[0] = z[:, :C // 2]
    z_ref[1] = z[:, C // 2:]
    craw = p[:, 0:4] * inv[:, None]
    coords_ref[...] = craw

    @pl.when(i == 0)
    def _():
        acc[...] = jnp.zeros_like(acc)

    acc[0, :] += jnp.sum(craw, axis=0)
    acc[1, :] += jnp.sum(craw * craw, axis=0)
    cstats_ref[...] = acc[...]
    dinv_ref[...] = dinv[:, None]


def _p1(psum, degp, gw0):
    return pl.pallas_call(
        _p1_body,
        grid=(GRID,),
        in_specs=[
            _row_spec(FW),
            pl.BlockSpec((NC, RB, 16), lambda i: (0, i, 0)),
            _full_spec((C, C)),
        ],
        out_specs=[
            _row_spec(4),
            _row_spec(1),
            _z_spec(),
            _full_spec((2, 4)),
        ],
        out_shape=[
            jax.ShapeDtypeStruct((N, 4), jnp.float32),
            jax.ShapeDtypeStruct((N, 1), jnp.float32),
            jax.ShapeDtypeStruct((NC, N, C // 2), jnp.float32),
            jax.ShapeDtypeStruct((2, 4), jnp.float32),
        ],
        scratch_shapes=[pltpu.VMEM((2, 4), jnp.float32)],
    )(psum, degp, gw0)


def _pa_body(agg_ref, z_ref, dinv_ref, gb_ref, t_ref, stats_ref, acc):
    i = pl.program_id(0)
    a = agg_ref[...]
    zz = z_ref[...]
    tl = jnp.concatenate([a[0] + zz[0], a[1] + zz[1]], axis=1)
    t = tl * dinv_ref[...] + gb_ref[0][None, :]
    t_ref[...] = t

    @pl.when(i == 0)
    def _():
        acc[...] = jnp.zeros_like(acc)

    acc[0, :] += jnp.sum(t, axis=0)
    acc[1, :] += jnp.sum(t * t, axis=0)
    stats_ref[...] = acc[...]


def _pa(agg, z, dinv, gb):
    return pl.pallas_call(
        _pa_body,
        grid=(GRID,),
        in_specs=[
            pl.BlockSpec((NC, RB, C // 2), lambda i: (0, i, 0)),
            _z_spec(),
            _row_spec(1),
            _full_spec((1, C)),
        ],
        out_specs=[_row_spec(C), _full_spec((2, C))],
        out_shape=[
            jax.ShapeDtypeStruct((N, C), jnp.float32),
            jax.ShapeDtypeStruct((2, C), jnp.float32),
        ],
        scratch_shapes=[pltpu.VMEM((2, C), jnp.float32)],
    )(agg, z, dinv, gb)


def _bn(x, stats, g, b):
    mu = stats[0] / N
    var = stats[1] / N - mu * mu
    return (x - mu[None, :]) * lax.rsqrt(var + EPS)[None, :] * g + b


def _pb_body(t_ref, stats_ref, g_ref, b_ref, psum_ref, mw_ref, mb_ref,
             u_ref, ustats_ref, acc, *, col):
    i = pl.program_id(0)
    h = jnp.maximum(_bn(t_ref[...], stats_ref[...], g_ref[0][None, :],
                        b_ref[0][None, :]), 0.0)
    p = psum_ref[...]
    inv = 1.0 / jnp.maximum(p[:, 4], 1.0)
    pool = p[:, col:col + C] * inv[:, None]
    x1 = jnp.concatenate([h, pool], axis=1)
    u = jnp.dot(x1, mw_ref[...], preferred_element_type=jnp.float32) \
        + mb_ref[0][None, :]
    u_ref[...] = u

    @pl.when(i == 0)
    def _():
        acc[...] = jnp.zeros_like(acc)

    acc[0, :] += jnp.sum(u, axis=0)
    acc[1, :] += jnp.sum(u * u, axis=0)
    ustats_ref[...] = acc[...]


def _pb(t, stats, g, b, psum, mw, mb, col):
    return pl.pallas_call(
        functools.partial(_pb_body, col=col),
        grid=(GRID,),
        in_specs=[
            _row_spec(C),
            _full_spec((2, C)),
            _full_spec((1, C)),
            _full_spec((1, C)),
            _row_spec(FW),
            _full_spec((2 * C, C)),
            _full_spec((1, C)),
        ],
        out_specs=[_row_spec(C), _full_spec((2, C))],
        out_shape=[
            jax.ShapeDtypeStruct((N, C), jnp.float32),
            jax.ShapeDtypeStruct((2, C), jnp.float32),
        ],
        scratch_shapes=[pltpu.VMEM((2, C), jnp.float32)],
    )(t, stats, g, b, psum, mw, mb)


def _pc_body(u_ref, stats_ref, g_ref, b_ref, gw_ref, dinv_ref, z_ref):
    y = _bn(u_ref[...], stats_ref[...], g_ref[0][None, :], b_ref[0][None, :])
    z = jnp.dot(y, gw_ref[...], preferred_element_type=jnp.float32) \
        * dinv_ref[...]
    z_ref[0] = z[:, :C // 2]
    z_ref[1] = z[:, C // 2:]


def _pc(u, stats, g, b, gw, dinv):
    return pl.pallas_call(
        _pc_body,
        grid=(GRID,),
        in_specs=[
            _row_spec(C),
            _full_spec((2, C)),
            _full_spec((1, C)),
            _full_spec((1, C)),
            _full_spec((C, C)),
            _row_spec(1),
        ],
        out_specs=_z_spec(),
        out_shape=jax.ShapeDtypeStruct((NC, N, C // 2), jnp.float32),
    )(u, stats, g, b, gw, dinv)


def _pd_body(t_ref, stats_ref, g_ref, b_ref, craw_ref, cstats_ref, pg_ref,
             pb_ref, mw_ref, mb_ref, u_ref, ustats_ref, acc):
    i = pl.program_id(0)
    h = jnp.maximum(_bn(t_ref[...], stats_ref[...], g_ref[0][None, :],
                        b_ref[0][None, :]), 0.0)
    cmu = cstats_ref[0] / N
    cvar = cstats_ref[1] / N - cmu * cmu
    cb = jnp.maximum((craw_ref[...] - cmu[None, :]) * lax.rsqrt(cvar + EPS)
                     * pg_ref[0][None, :] + pb_ref[0][None, :], 0.0)
    xc = jnp.concatenate([h, cb], axis=1)
    u = jnp.dot(xc, mw_ref[...], preferred_element_type=jnp.float32) \
        + mb_ref[0][None, :]
    u_ref[...] = u

    @pl.when(i == 0)
    def _():
        acc[...] = jnp.zeros_like(acc)

    acc[0, :] += jnp.sum(u, axis=0)
    acc[1, :] += jnp.sum(u * u, axis=0)
    ustats_ref[...] = acc[...]


def _pd(t, stats, g, b, craw, cstats, pg, pbb, mw, mb):
    return pl.pallas_call(
        _pd_body,
        grid=(GRID,),
        in_specs=[
            _row_spec(C),
            _full_spec((2, C)),
            _full_spec((1, C)),
            _full_spec((1, C)),
            _row_spec(4),
            _full_spec((2, 4)),
            _full_spec((1, 4)),
            _full_spec((1, 4)),
            _full_spec((C + 4, C)),
            _full_spec((1, C)),
        ],
        out_specs=[_row_spec(C), _full_spec((2, C))],
        out_shape=[
            jax.ShapeDtypeStruct((N, C), jnp.float32),
            jax.ShapeDtypeStruct((2, C), jnp.float32),
        ],
        scratch_shapes=[pltpu.VMEM((2, C), jnp.float32)],
    )(t, stats, g, b, craw, cstats, pg, pbb, mw, mb)


def _pe_body(u_ref, stats_ref, g_ref, b_ref, lw_ref, lb_ref, o_ref):
    y = _bn(u_ref[...], stats_ref[...], g_ref[0][None, :], b_ref[0][None, :])
    o_ref[...] = jnp.maximum(
        jnp.dot(y, lw_ref[...], preferred_element_type=jnp.float32)
        + lb_ref[0][None, :], 0.0)


def _pe(u, stats, g, b, lw, lb):
    return pl.pallas_call(
        _pe_body,
        grid=(GRID,),
        in_specs=[
            _row_spec(C),
            _full_spec((2, C)),
            _full_spec((1, C)),
            _full_spec((1, C)),
            _full_spec((C, 128)),
            _full_spec((1, 128)),
        ],
        out_specs=_row_spec(128),
        out_shape=jax.ShapeDtypeStruct((N, 128), jnp.float32),
    )(u, stats, g, b, lw, lb)


def kernel(fx, fy, skip0, skip1, skip2, params, labels, edges_nn):
    f32 = jnp.float32
    xx = jnp.broadcast_to(
        (jnp.arange(W, dtype=f32) / (W - 1))[:, None], (W, H)).reshape(-1)
    yy = jnp.broadcast_to(
        (jnp.arange(H, dtype=f32) / (H - 1))[None, :], (W, H)).reshape(-1)
    aux = jnp.concatenate([
        jnp.broadcast_to(jnp.stack([xx, yy], axis=-1)[None], (B, PIX, 2)),
        fx.reshape(B, PIX, 1).astype(f32),
        fy.reshape(B, PIX, 1).astype(f32),
        jnp.ones((B, PIX, 1), f32),
        jnp.zeros((B, PIX, AUX - 5), f32),
    ], axis=2)
    s0t, s1t, s2t = (sk.reshape(B, C, PIX).transpose(0, 2, 1).astype(f32)
                     for sk in (skip0, skip1, skip2))
    feat = jnp.stack([
        jnp.concatenate([aux, s0t, s1t[:, :, :FWH - AUX - C]], axis=2),
        jnp.concatenate([s1t[:, :, FWH - AUX - C:], s2t,
                         jnp.zeros((B, PIX, 2 * FWH - AUX - 3 * C), f32)],
                        axis=2),
    ])
    lbl = labels.reshape(B, PIX).astype(jnp.int32)

    src = jnp.pad(edges_nn[0].astype(jnp.int32), (0, EPAD - E))
    dst = jnp.pad(edges_nn[1].astype(jnp.int32), (0, EPAD - E))
    lab = jnp.pad(edges_nn[2].astype(jnp.int32), (0, EPAD - E),
                  constant_values=-1)

    p = params
    row = lambda a: a.reshape(1, -1).astype(f32)

    pool = _pool_kernel(feat, lbl)
    psum = jnp.concatenate([pool[0], pool[1]], axis=-1)[:, :S].reshape(N, FW)
    degp = _deg_kernel(dst, lab)
    craw, dinv, z0, cstats = _p1(psum, degp, p['gW0'].astype(f32))

    agg0 = _agg_kernel(z0.reshape(NC * N, C // 2), src, dst, lab)
    t0, st0 = _pa(agg0, z0, dinv, row(p['gb0']))
    u1, su1 = _pb(t0, st0, row(p['gbn_g0']), row(p['gbn_b0']), psum,
                  p['mW0'].astype(f32), row(p['mb0']), AUX + C)
    z1 = _pc(u1, su1, row(p['mbn_g0']), row(p['mbn_b0']),
             p['gW1'].astype(f32), dinv)

    agg1 = _agg_kernel(z1.reshape(NC * N, C // 2), src, dst, lab)
    t1, st1 = _pa(agg1, z1, dinv, row(p['gb1']))
    u2, su2 = _pb(t1, st1, row(p['gbn_g1']), row(p['gbn_b1']), psum,
                  p['mW1'].astype(f32), row(p['mb1']), AUX + 2 * C)
    z2 = _pc(u2, su2, row(p['mbn_g1']), row(p['mbn_b1']),
             p['gW2'].astype(f32), dinv)

    agg2 = _agg_kernel(z2.reshape(NC * N, C // 2), src, dst, lab)
    t2, st2 = _pa(agg2, z2, dinv, row(p['gb2']))
    u3, su3 = _pd(t2, st2, row(p['gbn_g2']), row(p['gbn_b2']), craw, cstats,
                  row(p['pre_g']), row(p['pre_b']),
                  p['mW2'].astype(f32), row(p['mb2']))
    return _pe(u3, su3, row(p['mbn_g2']), row(p['mbn_b2']),
               p['lW'].astype(f32), row(p['lb']))


# speedup vs baseline: 3.4551x; 1.2200x over previous
import functools

import jax
import jax.numpy as jnp
from jax import lax
from jax.experimental import pallas as pl
from jax.experimental.pallas import tpu as pltpu
from jax.experimental.pallas import tpu_sc as plsc

B, S, W, H = 4, 2500, 128, 128
PIX = W * H
N = B * S
E = 160000
C = 256
AUX = 16
FW = 800
FWH = FW // 2
NC, NS = 2, 16
EPAD = 163840
EW = 128
EWA = 80
NWIN = 128
PW = 64
SPAD = 2560
TROWS = 160
NPAD = N + 112
DROWS = NPAD // NS
RB = 1000
GRID = N // RB
EPS = 1e-5

_sc_mesh = plsc.VectorSubcoreMesh(core_axis_name="c", subcore_axis_name="s")


@functools.partial(
    pl.kernel,
    out_type=jax.ShapeDtypeStruct((NC, B, SPAD, FWH), jnp.float32),
    mesh=_sc_mesh,
    compiler_params=pltpu.CompilerParams(use_tc_tiling_on_sc=False),
    scratch_types=[
        pltpu.VMEM((PW, FWH), jnp.float32),
        pltpu.VMEM((PW, FWH), jnp.float32),
        pltpu.VMEM((B * 16, PW), jnp.int32),
        pltpu.VMEM_SHARED((SPAD, FWH), jnp.float32),
        pltpu.SemaphoreType.DMA,
        pltpu.SemaphoreType.DMA,
        pltpu.SemaphoreType.DMA,
        pltpu.SemaphoreType.DMA,
    ],
)
def _pool_kernel(feat, lbl, out, buf0, buf1, lbl2d, table, fs0, fs1, ss0, ss1):
    c = lax.axis_index("c")
    t = lax.axis_index("s")
    zf = jnp.zeros((16,), jnp.float32)
    bufs = (buf0, buf1)
    fsems = (fs0, fs1)
    ssems = (ss0, ss1)
    start = t * TROWS
    nwin = PIX // NS // PW

    def _zero_buf():
        def zrow(r, carry):
            for j in range(FWH // 16):
                buf0[r, pl.ds(16 * j, 16)] = zf
            return carry
        lax.fori_loop(0, PW, zrow, 0)

    def _zero_table():
        pltpu.sync_copy(buf0, table.at[pl.ds(start, PW)])
        pltpu.sync_copy(buf0, table.at[pl.ds(start + PW, PW)])
        pltpu.sync_copy(buf0.at[pl.ds(0, TROWS - 2 * PW)],
                        table.at[pl.ds(start + 2 * PW, TROWS - 2 * PW)])

    for img in range(B):
        pltpu.sync_copy(lbl.at[img, t], lbl2d.at[pl.ds(img * 16, 16)])
    _zero_buf()
    _zero_table()
    plsc.subcore_barrier()
    for img in range(B):
        base = t * (PIX // NS)
        cg = {}
        cs = {}
        cg[0] = pltpu.async_copy(feat.at[c, img, pl.ds(base, PW)], buf0, fs0)
        for w in range(nwin):
            b = w % 2
            cg[w].wait()
            cs[w] = pltpu.async_copy(bufs[b], table.at[lbl2d.at[img * 16 + w]],
                                     ssems[b], add=True)
            if w + 1 < nwin:
                if w >= 1:
                    cs[w - 1].wait()
                cg[w + 1] = pltpu.async_copy(
                    feat.at[c, img, pl.ds(base + (w + 1) * PW, PW)],
                    bufs[1 - b], fsems[1 - b])
        cs[nwin - 2].wait()
        cs[nwin - 1].wait()
        plsc.subcore_barrier()
        pltpu.sync_copy(table.at[pl.ds(start, TROWS)],
                        out.at[c, img, pl.ds(start, TROWS)])
        if img + 1 < B:
            plsc.subcore_barrier()
            _zero_buf()
            _zero_table()
            plsc.subcore_barrier()


@functools.partial(
    pl.kernel,
    out_type=jax.ShapeDtypeStruct((NC, NPAD, 16), jnp.float32),
    mesh=_sc_mesh,
    compiler_params=pltpu.CompilerParams(use_tc_tiling_on_sc=False),
    scratch_types=[
        pltpu.VMEM((EW, 16), jnp.float32),
        pltpu.VMEM((EW, 16), jnp.float32),
        pltpu.VMEM((EW,), jnp.int32),
        pltpu.VMEM((EW,), jnp.int32),
        pltpu.VMEM_SHARED((NPAD, 16), jnp.float32),
        pltpu.SemaphoreType.DMA,
    ],
)
def _deg_kernel(dst_h, lab_h, out, upd, zbuf, ibuf, lbuf, table, sem):
    c = lax.axis_index("c")
    t = lax.axis_index("s")
    zf = jnp.zeros((16,), jnp.float32)
    one0 = jnp.where(lax.iota(jnp.int32, 16) == 0, 1.0, 0.0)

    def zrow(r, carry):
        upd[r, :] = one0
        zbuf[r, :] = zf
        return carry
    lax.fori_loop(0, EW, zrow, 0)

    base_r = t * DROWS
    for k in range(4):
        pltpu.sync_copy(zbuf, table.at[pl.ds(base_r + EW * k, EW)])
    pltpu.sync_copy(zbuf.at[pl.ds(0, DROWS - 4 * EW)],
                    table.at[pl.ds(base_r + 4 * EW, DROWS - 4 * EW)])
    plsc.subcore_barrier()

    ept = EPAD // (NC * NS)
    ebase = (c * NS + t) * ept

    def win(w, carry):
        es = ebase + w * EW
        pltpu.sync_copy(dst_h.at[pl.ds(es, EW)], ibuf)
        pltpu.sync_copy(lab_h.at[pl.ds(es, EW)], lbuf)

        def cw(j, carry2):
            sl = pl.ds(16 * j, 16)
            d = ibuf[sl]
            l = lbuf[sl]
            ibuf[sl] = jnp.where(l != -1, d, N + t)
            return carry2
        lax.fori_loop(0, EW // 16, cw, 0)
        pltpu.async_copy(upd, table.at[ibuf], sem, add=True).wait()
        return carry
    lax.fori_loop(0, ept // EW, win, 0)
    plsc.subcore_barrier()
    pltpu.sync_copy(table.at[pl.ds(base_r, DROWS)],
                    out.at[c, pl.ds(base_r, DROWS)])


@functools.partial(
    pl.kernel,
    out_type=jax.ShapeDtypeStruct((NC, NPAD, C // 2), jnp.float32),
    mesh=_sc_mesh,
    compiler_params=pltpu.CompilerParams(use_tc_tiling_on_sc=False),
    scratch_types=[
        pltpu.VMEM((EWA, C // 2), jnp.float32),
        pltpu.VMEM((EWA, C // 2), jnp.float32),
        pltpu.VMEM((NWIN, EWA), jnp.int32),
        pltpu.VMEM((NWIN, EWA), jnp.int32),
        pltpu.VMEM_SHARED((NPAD, C // 2), jnp.float32),
        pltpu.SemaphoreType.DMA,
        pltpu.SemaphoreType.DMA,
        pltpu.SemaphoreType.DMA,
        pltpu.SemaphoreType.DMA,
    ],
)
def _agg_kernel(z, srcs_h, tgt_h, out, rows0, rows1, sidx, tidx,
                table, gs0, gs1, ss0, ss1):
    c = lax.axis_index("c")
    t = lax.axis_index("s")
    zf = jnp.zeros((16,), jnp.float32)

    def zrow(r, carry):
        for j in range(C // 2 // 16):
            rows0[r, pl.ds(16 * j, 16)] = zf
        return carry
    lax.fori_loop(0, EWA, zrow, 0)

    base_r = t * DROWS
    for k in range(7):
        pltpu.sync_copy(rows0, table.at[pl.ds(base_r + EWA * k, EWA)])
    pltpu.sync_copy(rows0.at[pl.ds(0, DROWS - 7 * EWA)],
                    table.at[pl.ds(base_r + 7 * EWA, DROWS - 7 * EWA)])
    pltpu.sync_copy(srcs_h.at[c, t], sidx)
    pltpu.sync_copy(tgt_h.at[t], tidx)
    plsc.subcore_barrier()

    def g(w, rows, sem):
        return pltpu.async_copy(z.at[sidx.at[w]], rows, sem)

    def s(w, rows, sem):
        return pltpu.async_copy(rows, table.at[tidx.at[w]], sem, add=True)

    def gwait(w, rows, sem):
        pltpu.make_async_copy(z.at[sidx.at[w]], rows, sem).wait()

    def swait(w, rows, sem):
        pltpu.make_async_copy(rows, table.at[tidx.at[w]], sem).wait()

    g(0, rows0, gs0)
    g(1, rows1, gs1)

    def body(p, carry):
        w0 = 2 * p
        gwait(w0, rows0, gs0)
        s(w0, rows0, ss0)
        gwait(w0 + 1, rows1, gs1)
        swait(w0, rows0, ss0)

        @pl.when(p < NWIN // 2 - 1)
        def _():
            g(w0 + 2, rows0, gs0)
        s(w0 + 1, rows1, ss1)
        swait(w0 + 1, rows1, ss1)

        @pl.when(p < NWIN // 2 - 1)
        def _():
            g(w0 + 3, rows1, gs1)
        return carry
    lax.fori_loop(0, NWIN // 2, body, 0)
    plsc.subcore_barrier()
    pltpu.sync_copy(table.at[pl.ds(base_r, DROWS)],
                    out.at[c, pl.ds(base_r, DROWS)])


def _row_spec(cols):
    return pl.BlockSpec((RB, cols), lambda i: (i, 0))


def _full_spec(shape):
    nd = len(shape)
    return pl.BlockSpec(shape, lambda i: (0,) * nd)


def _z_spec():
    return pl.BlockSpec((NC, RB, C // 2), lambda i: (0, i, 0))


def _p1_body(psum_ref, degp_ref, gw_ref, coords_ref, dinv_ref, z_ref,
             cstats_ref, acc):
    i = pl.program_id(0)
    p = psum_ref[...]
    cnt = p[:, 4]
    inv = 1.0 / jnp.maximum(cnt, 1.0)
    x0 = p[:, AUX:AUX + C] * inv[:, None]
    xw = jnp.dot(x0, gw_ref[...], preferred_element_type=jnp.float32)
    degp = degp_ref[...]
    deg = degp[0, :, 0] + degp[1, :, 0] + 1.0
    dinv = lax.rsqrt(deg)
    z = xw * dinv[:, None]
    z_ref[0] = z[:, :C // 2]
    z_ref[1] = z[:, C // 2:]
    craw = p[:, 0:4] * inv[:, None]
    coords_ref[...] = craw

    @pl.when(i == 0)
    def _():
        acc[...] = jnp.zeros_like(acc)

    acc[0, :] += jnp.sum(craw, axis=0)
    acc[1, :] += jnp.sum(craw * craw, axis=0)
    cstats_ref[...] = acc[...]
    dinv_ref[...] = dinv[:, None]


def _p1(psum, degp, gw0):
    return pl.pallas_call(
        _p1_body,
        grid=(GRID,),
        in_specs=[
            _row_spec(FW),
            pl.BlockSpec((NC, RB, 16), lambda i: (0, i, 0)),
            _full_spec((C, C)),
        ],
        out_specs=[
            _row_spec(4),
            _row_spec(1),
            _z_spec(),
            _full_spec((2, 4)),
        ],
        out_shape=[
            jax.ShapeDtypeStruct((N, 4), jnp.float32),
            jax.ShapeDtypeStruct((N, 1), jnp.float32),
            jax.ShapeDtypeStruct((NC, N, C // 2), jnp.float32),
            jax.ShapeDtypeStruct((2, 4), jnp.float32),
        ],
        scratch_shapes=[pltpu.VMEM((2, 4), jnp.float32)],
    )(psum, degp, gw0)


def _pa_body(agg_ref, z_ref, dinv_ref, gb_ref, t_ref, stats_ref, acc):
    i = pl.program_id(0)
    a = agg_ref[...]
    zz = z_ref[...]
    tl = jnp.concatenate([a[0] + zz[0], a[1] + zz[1]], axis=1)
    t = tl * dinv_ref[...] + gb_ref[0][None, :]
    t_ref[...] = t

    @pl.when(i == 0)
    def _():
        acc[...] = jnp.zeros_like(acc)

    acc[0, :] += jnp.sum(t, axis=0)
    acc[1, :] += jnp.sum(t * t, axis=0)
    stats_ref[...] = acc[...]


def _pa(agg, z, dinv, gb):
    return pl.pallas_call(
        _pa_body,
        grid=(GRID,),
        in_specs=[
            pl.BlockSpec((NC, RB, C // 2), lambda i: (0, i, 0)),
            _z_spec(),
            _row_spec(1),
            _full_spec((1, C)),
        ],
        out_specs=[_row_spec(C), _full_spec((2, C))],
        out_shape=[
            jax.ShapeDtypeStruct((N, C), jnp.float32),
            jax.ShapeDtypeStruct((2, C), jnp.float32),
        ],
        scratch_shapes=[pltpu.VMEM((2, C), jnp.float32)],
    )(agg, z, dinv, gb)


def _bn(x, stats, g, b):
    mu = stats[0] / N
    var = stats[1] / N - mu * mu
    return (x - mu[None, :]) * lax.rsqrt(var + EPS)[None, :] * g + b


def _pb_body(t_ref, stats_ref, g_ref, b_ref, psum_ref, mw_ref, mb_ref,
             u_ref, ustats_ref, acc, *, col):
    i = pl.program_id(0)
    h = jnp.maximum(_bn(t_ref[...], stats_ref[...], g_ref[0][None, :],
                        b_ref[0][None, :]), 0.0)
    p = psum_ref[...]
    inv = 1.0 / jnp.maximum(p[:, 4], 1.0)
    pool = p[:, col:col + C] * inv[:, None]
    x1 = jnp.concatenate([h, pool], axis=1)
    u = jnp.dot(x1, mw_ref[...], preferred_element_type=jnp.float32) \
        + mb_ref[0][None, :]
    u_ref[...] = u

    @pl.when(i == 0)
    def _():
        acc[...] = jnp.zeros_like(acc)

    acc[0, :] += jnp.sum(u, axis=0)
    acc[1, :] += jnp.sum(u * u, axis=0)
    ustats_ref[...] = acc[...]


def _pb(t, stats, g, b, psum, mw, mb, col):
    return pl.pallas_call(
        functools.partial(_pb_body, col=col),
        grid=(GRID,),
        in_specs=[
            _row_spec(C),
            _full_spec((2, C)),
            _full_spec((1, C)),
            _full_spec((1, C)),
            _row_spec(FW),
            _full_spec((2 * C, C)),
            _full_spec((1, C)),
        ],
        out_specs=[_row_spec(C), _full_spec((2, C))],
        out_shape=[
            jax.ShapeDtypeStruct((N, C), jnp.float32),
            jax.ShapeDtypeStruct((2, C), jnp.float32),
        ],
        scratch_shapes=[pltpu.VMEM((2, C), jnp.float32)],
    )(t, stats, g, b, psum, mw, mb)


def _pc_body(u_ref, stats_ref, g_ref, b_ref, gw_ref, dinv_ref, z_ref):
    y = _bn(u_ref[...], stats_ref[...], g_ref[0][None, :], b_ref[0][None, :])
    z = jnp.dot(y, gw_ref[...], preferred_element_type=jnp.float32) \
        * dinv_ref[...]
    z_ref[0] = z[:, :C // 2]
    z_ref[1] = z[:, C // 2:]


def _pc(u, stats, g, b, gw, dinv):
    return pl.pallas_call(
        _pc_body,
        grid=(GRID,),
        in_specs=[
            _row_spec(C),
            _full_spec((2, C)),
            _full_spec((1, C)),
            _full_spec((1, C)),
            _full_spec((C, C)),
            _row_spec(1),
        ],
        out_specs=_z_spec(),
        out_shape=jax.ShapeDtypeStruct((NC, N, C // 2), jnp.float32),
    )(u, stats, g, b, gw, dinv)


def _pd_body(t_ref, stats_ref, g_ref, b_ref, craw_ref, cstats_ref, pg_ref,
             pb_ref, mw_ref, mb_ref, u_ref, ustats_ref, acc):
    i = pl.program_id(0)
    h = jnp.maximum(_bn(t_ref[...], stats_ref[...], g_ref[0][None, :],
                        b_ref[0][None, :]), 0.0)
    cmu = cstats_ref[0] / N
    cvar = cstats_ref[1] / N - cmu * cmu
    cb = jnp.maximum((craw_ref[...] - cmu[None, :]) * lax.rsqrt(cvar + EPS)
                     * pg_ref[0][None, :] + pb_ref[0][None, :], 0.0)
    xc = jnp.concatenate([h, cb], axis=1)
    u = jnp.dot(xc, mw_ref[...], preferred_element_type=jnp.float32) \
        + mb_ref[0][None, :]
    u_ref[...] = u

    @pl.when(i == 0)
    def _():
        acc[...] = jnp.zeros_like(acc)

    acc[0, :] += jnp.sum(u, axis=0)
    acc[1, :] += jnp.sum(u * u, axis=0)
    ustats_ref[...] = acc[...]


def _pd(t, stats, g, b, craw, cstats, pg, pbb, mw, mb):
    return pl.pallas_call(
        _pd_body,
        grid=(GRID,),
        in_specs=[
            _row_spec(C),
            _full_spec((2, C)),
            _full_spec((1, C)),
            _full_spec((1, C)),
            _row_spec(4),
            _full_spec((2, 4)),
            _full_spec((1, 4)),
            _full_spec((1, 4)),
            _full_spec((C + 4, C)),
            _full_spec((1, C)),
        ],
        out_specs=[_row_spec(C), _full_spec((2, C))],
        out_shape=[
            jax.ShapeDtypeStruct((N, C), jnp.float32),
            jax.ShapeDtypeStruct((2, C), jnp.float32),
        ],
        scratch_shapes=[pltpu.VMEM((2, C), jnp.float32)],
    )(t, stats, g, b, craw, cstats, pg, pbb, mw, mb)


def _pe_body(u_ref, stats_ref, g_ref, b_ref, lw_ref, lb_ref, o_ref):
    y = _bn(u_ref[...], stats_ref[...], g_ref[0][None, :], b_ref[0][None, :])
    o_ref[...] = jnp.maximum(
        jnp.dot(y, lw_ref[...], preferred_element_type=jnp.float32)
        + lb_ref[0][None, :], 0.0)


def _pe(u, stats, g, b, lw, lb):
    return pl.pallas_call(
        _pe_body,
        grid=(GRID,),
        in_specs=[
            _row_spec(C),
            _full_spec((2, C)),
            _full_spec((1, C)),
            _full_spec((1, C)),
            _full_spec((C, 128)),
            _full_spec((1, 128)),
        ],
        out_specs=_row_spec(128),
        out_shape=jax.ShapeDtypeStruct((N, 128), jnp.float32),
    )(u, stats, g, b, lw, lb)


def _eprep_body(s_ref, d_ref, l_ref, srcs_ref, tgt_ref):
    t = pl.program_id(0)
    s = s_ref[...]
    d = d_ref[...]
    l = l_ref[...]
    tgt_ref[...] = jnp.where(l != -1, d, N + t)
    srcs_ref[0] = s
    srcs_ref[1] = s + N


def _eprep(src, dst, lab):
    epw = EPAD // NS
    return pl.pallas_call(
        _eprep_body,
        grid=(NS,),
        in_specs=[pl.BlockSpec((1, 1, epw), lambda t: (t, 0, 0))] * 3,
        out_specs=[
            pl.BlockSpec((NC, 1, 1, epw), lambda t: (0, t, 0, 0)),
            pl.BlockSpec((1, 1, epw), lambda t: (t, 0, 0)),
        ],
        out_shape=[
            jax.ShapeDtypeStruct((NC, NS, 1, epw), jnp.int32),
            jax.ShapeDtypeStruct((NS, 1, epw), jnp.int32),
        ],
    )(src.reshape(NS, 1, epw), dst.reshape(NS, 1, epw),
      lab.reshape(NS, 1, epw))


def kernel(fx, fy, skip0, skip1, skip2, params, labels, edges_nn):
    f32 = jnp.float32
    xx = jnp.broadcast_to(
        (jnp.arange(W, dtype=f32) / (W - 1))[:, None], (W, H)).reshape(-1)
    yy = jnp.broadcast_to(
        (jnp.arange(H, dtype=f32) / (H - 1))[None, :], (W, H)).reshape(-1)
    aux = jnp.concatenate([
        jnp.broadcast_to(jnp.stack([xx, yy], axis=-1)[None], (B, PIX, 2)),
        fx.reshape(B, PIX, 1).astype(f32),
        fy.reshape(B, PIX, 1).astype(f32),
        jnp.ones((B, PIX, 1), f32),
        jnp.zeros((B, PIX, AUX - 5), f32),
    ], axis=2)
    s0t, s1t, s2t = (sk.reshape(B, C, PIX).transpose(0, 2, 1).astype(f32)
                     for sk in (skip0, skip1, skip2))
    feat = jnp.stack([
        jnp.concatenate([aux, s0t, s1t[:, :, :FWH - AUX - C]], axis=2),
        jnp.concatenate([s1t[:, :, FWH - AUX - C:], s2t,
                         jnp.zeros((B, PIX, 2 * FWH - AUX - 3 * C), f32)],
                        axis=2),
    ])
    lbl = labels.reshape(B, PIX).astype(jnp.int32)

    src = jnp.pad(edges_nn[0].astype(jnp.int32), (0, EPAD - E))
    dst = jnp.pad(edges_nn[1].astype(jnp.int32), (0, EPAD - E))
    lab = jnp.pad(edges_nn[2].astype(jnp.int32), (0, EPAD - E),
                  constant_values=-1)

    p = params
    row = lambda a: a.reshape(1, -1).astype(f32)

    pool = _pool_kernel(feat, lbl.reshape(B, NS, 16, PW))
    psum = jnp.concatenate([pool[0], pool[1]], axis=-1)[:, :S].reshape(N, FW)
    srcs4, tgt3 = _eprep(src, dst, lab)
    srcs4 = srcs4.reshape(NC, NS, NWIN, EWA)
    tgt3 = tgt3.reshape(NS, NWIN, EWA)
    degp = _deg_kernel(dst, lab)
    craw, dinv, z0, cstats = _p1(psum, degp, p['gW0'].astype(f32))

    agg0 = _agg_kernel(z0.reshape(NC * N, C // 2), srcs4, tgt3)
    t0, st0 = _pa(agg0, z0, dinv, row(p['gb0']))
    u1, su1 = _pb(t0, st0, row(p['gbn_g0']), row(p['gbn_b0']), psum,
                  p['mW0'].astype(f32), row(p['mb0']), AUX + C)
    z1 = _pc(u1, su1, row(p['mbn_g0']), row(p['mbn_b0']),
             p['gW1'].astype(f32), dinv)

    agg1 = _agg_kernel(z1.reshape(NC * N, C // 2), srcs4, tgt3)
    t1, st1 = _pa(agg1, z1, dinv, row(p['gb1']))
    u2, su2 = _pb(t1, st1, row(p['gbn_g1']), row(p['gbn_b1']), psum,
                  p['mW1'].astype(f32), row(p['mb1']), AUX + 2 * C)
    z2 = _pc(u2, su2, row(p['mbn_g1']), row(p['mbn_b1']),
             p['gW2'].astype(f32), dinv)

    agg2 = _agg_kernel(z2.reshape(NC * N, C // 2), srcs4, tgt3)
    t2, st2 = _pa(agg2, z2, dinv, row(p['gb2']))
    u3, su3 = _pd(t2, st2, row(p['gbn_g2']), row(p['gbn_b2']), craw, cstats,
                  row(p['pre_g']), row(p['pre_b']),
                  p['mW2'].astype(f32), row(p['mb2']))
    return _pe(u3, su3, row(p['mbn_g2']), row(p['mbn_b2']),
               p['lW'].astype(f32), row(p['lb']))


# speedup vs baseline: 4.0827x; 1.1816x over previous
import functools

import jax
import jax.numpy as jnp
from jax import lax
from jax.experimental import pallas as pl
from jax.experimental.pallas import tpu as pltpu
from jax.experimental.pallas import tpu_sc as plsc

B, S, W, H = 4, 2500, 128, 128
PIX = W * H
N = B * S
E = 160000
C = 256
AUX = 16
FW = 800
FWH = FW // 2
NC, NS = 2, 16
EPAD = 163840
EWA = 80
NWIN = 128
PW = 64
SPAD = 2560
TROWS = 160
NP = B * SPAD
NPAD2 = 10368
DROWS = NPAD2 // NS
RB = 1024
GRID = NP // RB
PB = 2048
EPS = 1e-5

_sc_mesh = plsc.VectorSubcoreMesh(core_axis_name="c", subcore_axis_name="s")
_sc_params = pltpu.CompilerParams(use_tc_tiling_on_sc=False)


def _fasm_body(s0_ref, s1_ref, s2_ref, fx_ref, fy_ref, out_ref):
    f32 = jnp.float32
    j = pl.program_id(1)
    t0 = jnp.transpose(s0_ref[0])
    t1 = jnp.transpose(s1_ref[0])
    t2 = jnp.transpose(s2_ref[0])
    p = j * PB + lax.broadcasted_iota(jnp.int32, (PB, 1), 0)
    xxv = (p // H).astype(f32) / (W - 1)
    yyv = (p % H).astype(f32) / (H - 1)
    fxv = fx_ref[0, 0][:, None]
    fyv = fy_ref[0, 0][:, None]
    ones = jnp.ones((PB, 1), f32)
    zpad = jnp.zeros((PB, AUX - 5), f32)
    k1 = FWH - AUX - C
    out_ref[0, 0] = jnp.concatenate(
        [xxv, yyv, fxv, fyv, ones, zpad, t0, t1[:, :k1]], axis=1)
    out_ref[1, 0] = jnp.concatenate(
        [t1[:, k1:], t2, jnp.zeros((PB, 2 * FWH - AUX - 3 * C), f32)], axis=1)


def _fasm(s0, s1, s2, fx3, fy3):
    return pl.pallas_call(
        _fasm_body,
        grid=(B, PIX // PB),
        in_specs=[pl.BlockSpec((1, C, PB), lambda b, j: (b, 0, j))] * 3
        + [pl.BlockSpec((1, 1, PB), lambda b, j: (b, 0, j))] * 2,
        out_specs=pl.BlockSpec((NC, 1, PB, FWH), lambda b, j: (0, b, j, 0)),
        out_shape=jax.ShapeDtypeStruct((NC, B, PIX, FWH), jnp.float32),
    )(s0, s1, s2, fx3, fy3)


def _eprep_body(s_ref, d_ref, l_ref, srcs_ref, tgt_ref):
    t = pl.program_id(0)
    s = s_ref[...]
    d = d_ref[...]
    l = l_ref[...]
    srm = s + 60 * (s // S)
    drm = d + 60 * (d // S)
    tgt_ref[...] = jnp.where(l != -1, drm, NP + t)
    srcs_ref[0] = srm
    srcs_ref[1] = srm + NP


def _eprep(src, dst, lab):
    epw = EPAD // NS
    return pl.pallas_call(
        _eprep_body,
        grid=(NS,),
        in_specs=[pl.BlockSpec((1, 1, epw), lambda t: (t, 0, 0))] * 3,
        out_specs=[
            pl.BlockSpec((NC, 1, 1, epw), lambda t: (0, t, 0, 0)),
            pl.BlockSpec((1, 1, epw), lambda t: (t, 0, 0)),
        ],
        out_shape=[
            jax.ShapeDtypeStruct((NC, NS, 1, epw), jnp.int32),
            jax.ShapeDtypeStruct((NS, 1, epw), jnp.int32),
        ],
    )(src.reshape(NS, 1, epw), dst.reshape(NS, 1, epw),
      lab.reshape(NS, 1, epw))


@functools.partial(
    pl.kernel,
    out_type=jax.ShapeDtypeStruct((NC, B, SPAD, FWH), jnp.float32),
    mesh=_sc_mesh,
    compiler_params=_sc_params,
    scratch_types=[
        pltpu.VMEM((PW, FWH), jnp.float32),
        pltpu.VMEM((PW, FWH), jnp.float32),
        pltpu.VMEM((B * 16, PW), jnp.int32),
        pltpu.VMEM_SHARED((SPAD, FWH), jnp.float32),
        pltpu.SemaphoreType.DMA,
        pltpu.SemaphoreType.DMA,
        pltpu.SemaphoreType.DMA,
        pltpu.SemaphoreType.DMA,
    ],
)
def _pool_kernel(feat, lbl, out, buf0, buf1, lbl2d, table, fs0, fs1, ss0, ss1):
    c = lax.axis_index("c")
    t = lax.axis_index("s")
    zf = jnp.zeros((16,), jnp.float32)
    bufs = (buf0, buf1)
    fsems = (fs0, fs1)
    ssems = (ss0, ss1)
    start = t * TROWS
    nwin = PIX // NS // PW

    def _zero_buf():
        def zrow(r, carry):
            for j in range(FWH // 16):
                buf0[r, pl.ds(16 * j, 16)] = zf
            return carry
        lax.fori_loop(0, PW, zrow, 0)

    def _zero_table():
        pltpu.sync_copy(buf0, table.at[pl.ds(start, PW)])
        pltpu.sync_copy(buf0, table.at[pl.ds(start + PW, PW)])
        pltpu.sync_copy(buf0.at[pl.ds(0, TROWS - 2 * PW)],
                        table.at[pl.ds(start + 2 * PW, TROWS - 2 * PW)])

    for img in range(B):
        pltpu.sync_copy(lbl.at[img, t], lbl2d.at[pl.ds(img * 16, 16)])
    _zero_buf()
    _zero_table()
    plsc.subcore_barrier()
    for img in range(B):
        base = t * (PIX // NS)
        cg = {}
        cs = {}
        cg[0] = pltpu.async_copy(feat.at[c, img, pl.ds(base, PW)], buf0, fs0)
        for w in range(nwin):
            b = w % 2
            cg[w].wait()
            cs[w] = pltpu.async_copy(bufs[b], table.at[lbl2d.at[img * 16 + w]],
                                     ssems[b], add=True)
            if w + 1 < nwin:
                if w >= 1:
                    cs[w - 1].wait()
                cg[w + 1] = pltpu.async_copy(
                    feat.at[c, img, pl.ds(base + (w + 1) * PW, PW)],
                    bufs[1 - b], fsems[1 - b])
        cs[nwin - 2].wait()
        cs[nwin - 1].wait()
        plsc.subcore_barrier()
        pltpu.sync_copy(table.at[pl.ds(start, TROWS)],
                        out.at[c, img, pl.ds(start, TROWS)])
        if img + 1 < B:
            plsc.subcore_barrier()
            _zero_buf()
            _zero_table()
            plsc.subcore_barrier()


@functools.partial(
    pl.kernel,
    out_type=jax.ShapeDtypeStruct((NC, NPAD2, 16), jnp.float32),
    mesh=_sc_mesh,
    compiler_params=_sc_params,
    scratch_types=[
        pltpu.VMEM((EWA, 16), jnp.float32),
        pltpu.VMEM((EWA, 16), jnp.float32),
        pltpu.VMEM((NWIN // 2, EWA), jnp.int32),
        pltpu.VMEM_SHARED((NPAD2, 16), jnp.float32),
        pltpu.SemaphoreType.DMA,
    ],
)
def _deg_kernel(tgt_h, out, upd, zbuf, tidx, table, sem):
    c = lax.axis_index("c")
    t = lax.axis_index("s")
    zf = jnp.zeros((16,), jnp.float32)
    one0 = jnp.where(lax.iota(jnp.int32, 16) == 0, 1.0, 0.0)

    def zrow(r, carry):
        upd[r, :] = one0
        zbuf[r, :] = zf
        return carry
    lax.fori_loop(0, EWA, zrow, 0)

    base_r = t * DROWS
    for k in range(8):
        pltpu.sync_copy(zbuf, table.at[pl.ds(base_r + EWA * k, EWA)])
    pltpu.sync_copy(zbuf.at[pl.ds(0, DROWS - 8 * EWA)],
                    table.at[pl.ds(base_r + 8 * EWA, DROWS - 8 * EWA)])
    pltpu.sync_copy(tgt_h.at[t, pl.ds(c * (NWIN // 2), NWIN // 2)], tidx)
    plsc.subcore_barrier()

    def win(w, carry):
        pltpu.async_copy(upd, table.at[tidx.at[w]], sem, add=True)
        return carry
    lax.fori_loop(0, NWIN // 2, win, 0)

    def drain(w, carry):
        pltpu.make_async_copy(upd, table.at[tidx.at[w]], sem).wait()
        return carry
    lax.fori_loop(0, NWIN // 2, drain, 0)
    plsc.subcore_barrier()
    pltpu.sync_copy(table.at[pl.ds(base_r, DROWS)],
                    out.at[c, pl.ds(base_r, DROWS)])


@functools.partial(
    pl.kernel,
    out_type=jax.ShapeDtypeStruct((NC, NPAD2, C // 2), jnp.float32),
    mesh=_sc_mesh,
    compiler_params=_sc_params,
    scratch_types=[
        pltpu.VMEM((EWA, C // 2), jnp.float32),
        pltpu.VMEM((EWA, C // 2), jnp.float32),
        pltpu.VMEM((NWIN, EWA), jnp.int32),
        pltpu.VMEM((NWIN, EWA), jnp.int32),
        pltpu.VMEM_SHARED((NPAD2, C // 2), jnp.float32),
        pltpu.SemaphoreType.DMA,
        pltpu.SemaphoreType.DMA,
        pltpu.SemaphoreType.DMA,
        pltpu.SemaphoreType.DMA,
    ],
)
def _agg_kernel(z, srcs_h, tgt_h, out, rows0, rows1, sidx, tidx,
                table, gs0, gs1, ss0, ss1):
    c = lax.axis_index("c")
    t = lax.axis_index("s")
    zf = jnp.zeros((16,), jnp.float32)

    def zrow(r, carry):
        for j in range(C // 2 // 16):
            rows0[r, pl.ds(16 * j, 16)] = zf
        return carry
    lax.fori_loop(0, EWA, zrow, 0)

    base_r = t * DROWS
    for k in range(8):
        pltpu.sync_copy(rows0, table.at[pl.ds(base_r + EWA * k, EWA)])
    pltpu.sync_copy(rows0.at[pl.ds(0, DROWS - 8 * EWA)],
                    table.at[pl.ds(base_r + 8 * EWA, DROWS - 8 * EWA)])
    pltpu.sync_copy(srcs_h.at[c, t], sidx)
    pltpu.sync_copy(tgt_h.at[t], tidx)
    plsc.subcore_barrier()

    def g(w, rows, sem):
        return pltpu.async_copy(z.at[sidx.at[w]], rows, sem)

    def s(w, rows, sem):
        return pltpu.async_copy(rows, table.at[tidx.at[w]], sem, add=True)

    def gwait(w, rows, sem):
        pltpu.make_async_copy(z.at[sidx.at[w]], rows, sem).wait()

    def swait(w, rows, sem):
        pltpu.make_async_copy(rows, table.at[tidx.at[w]], sem).wait()

    g(0, rows0, gs0)
    g(1, rows1, gs1)

    def body(p, carry):
        w0 = 2 * p
        gwait(w0, rows0, gs0)
        s(w0, rows0, ss0)
        gwait(w0 + 1, rows1, gs1)
        swait(w0, rows0, ss0)

        @pl.when(p < NWIN // 2 - 1)
        def _():
            g(w0 + 2, rows0, gs0)
        s(w0 + 1, rows1, ss1)
        swait(w0 + 1, rows1, ss1)

        @pl.when(p < NWIN // 2 - 1)
        def _():
            g(w0 + 3, rows1, gs1)
        return carry
    lax.fori_loop(0, NWIN // 2, body, 0)
    plsc.subcore_barrier()
    pltpu.sync_copy(table.at[pl.ds(base_r, DROWS)],
                    out.at[c, pl.ds(base_r, DROWS)])


def _row_spec(cols):
    return pl.BlockSpec((RB, cols), lambda i: (i, 0))


def _full_spec(shape):
    nd = len(shape)
    return pl.BlockSpec(shape, lambda i: (0,) * nd)


def _z_spec():
    return pl.BlockSpec((NC, RB, C // 2), lambda i: (0, i, 0))


def _psum_spec():
    return pl.BlockSpec((NC, RB, FWH), lambda i: (0, i, 0))


def _row_mask(i):
    r = i * RB + lax.broadcasted_iota(jnp.int32, (RB, 1), 0)
    return (r - (r // SPAD) * SPAD) < S


def _p1_body(psum_ref, degp_ref, gw_ref, coords_ref, dinv_ref, z_ref,
             cstats_ref, acc):
    i = pl.program_id(0)
    p0 = psum_ref[0]
    cnt = p0[:, 4]
    inv = 1.0 / jnp.maximum(cnt, 1.0)
    x0 = p0[:, AUX:AUX + C] * inv[:, None]
    xw = jnp.dot(x0, gw_ref[...], preferred_element_type=jnp.float32)
    degp = degp_ref[...]
    deg = degp[0, :, 0] + degp[1, :, 0] + 1.0
    dinv = lax.rsqrt(deg)
    z = xw * dinv[:, None]
    z_ref[0] = z[:, :C // 2]
    z_ref[1] = z[:, C // 2:]
    craw = p0[:, 0:4] * inv[:, None]
    coords_ref[...] = craw

    @pl.when(i == 0)
    def _():
        acc[...] = jnp.zeros_like(acc)

    cm = jnp.where(_row_mask(i), craw, 0.0)
    acc[0, :] += jnp.sum(cm, axis=0)
    acc[1, :] += jnp.sum(cm * craw, axis=0)
    cstats_ref[...] = acc[...]
    dinv_ref[...] = dinv[:, None]


def _p1(psum, degp, gw0):
    return pl.pallas_call(
        _p1_body,
        grid=(GRID,),
        in_specs=[
            _psum_spec(),
            pl.BlockSpec((NC, RB, 16), lambda i: (0, i, 0)),
            _full_spec((C, C)),
        ],
        out_specs=[
            _row_spec(4),
            _row_spec(1),
            _z_spec(),
            _full_spec((2, 4)),
        ],
        out_shape=[
            jax.ShapeDtypeStruct((NP, 4), jnp.float32),
            jax.ShapeDtypeStruct((NP, 1), jnp.float32),
            jax.ShapeDtypeStruct((NC, NP, C // 2), jnp.float32),
            jax.ShapeDtypeStruct((2, 4), jnp.float32),
        ],
        scratch_shapes=[pltpu.VMEM((2, 4), jnp.float32)],
    )(psum, degp, gw0)


def _pa_body(agg_ref, z_ref, dinv_ref, gb_ref, t_ref, stats_ref, acc):
    i = pl.program_id(0)
    a = agg_ref[...]
    zz = z_ref[...]
    tl = jnp.concatenate([a[0] + zz[0], a[1] + zz[1]], axis=1)
    t = tl * dinv_ref[...] + gb_ref[0][None, :]
    t_ref[...] = t

    @pl.when(i == 0)
    def _():
        acc[...] = jnp.zeros_like(acc)

    tm = jnp.where(_row_mask(i), t, 0.0)
    acc[0, :] += jnp.sum(tm, axis=0)
    acc[1, :] += jnp.sum(tm * t, axis=0)
    stats_ref[...] = acc[...]


def _pa(agg, z, dinv, gb):
    return pl.pallas_call(
        _pa_body,
        grid=(GRID,),
        in_specs=[
            pl.BlockSpec((NC, RB, C // 2), lambda i: (0, i, 0)),
            _z_spec(),
            _row_spec(1),
            _full_spec((1, C)),
        ],
        out_specs=[_row_spec(C), _full_spec((2, C))],
        out_shape=[
            jax.ShapeDtypeStruct((NP, C), jnp.float32),
            jax.ShapeDtypeStruct((2, C), jnp.float32),
        ],
        scratch_shapes=[pltpu.VMEM((2, C), jnp.float32)],
    )(agg, z, dinv, gb)


def _bn(x, stats, g, b):
    mu = stats[0] / N
    var = stats[1] / N - mu * mu
    return (x - mu[None, :]) * lax.rsqrt(var + EPS)[None, :] * g + b


def _pb_body(t_ref, stats_ref, g_ref, b_ref, psum_ref, mw_ref, mb_ref,
             u_ref, ustats_ref, acc, *, half):
    i = pl.program_id(0)
    h = jnp.maximum(_bn(t_ref[...], stats_ref[...], g_ref[0][None, :],
                        b_ref[0][None, :]), 0.0)
    p0 = psum_ref[0]
    p1 = psum_ref[1]
    inv = 1.0 / jnp.maximum(p0[:, 4], 1.0)
    k1 = FWH - AUX - C
    if half == 0:
        pool = jnp.concatenate([p0[:, AUX + C:], p1[:, :C - k1]], axis=1) \
            * inv[:, None]
    else:
        pool = p1[:, k1:k1 + C] * inv[:, None]
    x1 = jnp.concatenate([h, pool], axis=1)
    u = jnp.dot(x1, mw_ref[...], preferred_element_type=jnp.float32) \
        + mb_ref[0][None, :]
    u_ref[...] = u

    @pl.when(i == 0)
    def _():
        acc[...] = jnp.zeros_like(acc)

    um = jnp.where(_row_mask(i), u, 0.0)
    acc[0, :] += jnp.sum(um, axis=0)
    acc[1, :] += jnp.sum(um * u, axis=0)
    ustats_ref[...] = acc[...]


def _pb(t, stats, g, b, psum, mw, mb, half):
    return pl.pallas_call(
        functools.partial(_pb_body, half=half),
        grid=(GRID,),
        in_specs=[
            _row_spec(C),
            _full_spec((2, C)),
            _full_spec((1, C)),
            _full_spec((1, C)),
            _psum_spec(),
            _full_spec((2 * C, C)),
            _full_spec((1, C)),
        ],
        out_specs=[_row_spec(C), _full_spec((2, C))],
        out_shape=[
            jax.ShapeDtypeStruct((NP, C), jnp.float32),
            jax.ShapeDtypeStruct((2, C), jnp.float32),
        ],
        scratch_shapes=[pltpu.VMEM((2, C), jnp.float32)],
    )(t, stats, g, b, psum, mw, mb)


def _pc_body(u_ref, stats_ref, g_ref, b_ref, gw_ref, dinv_ref, z_ref):
    y = _bn(u_ref[...], stats_ref[...], g_ref[0][None, :], b_ref[0][None, :])
    z = jnp.dot(y, gw_ref[...], preferred_element_type=jnp.float32) \
        * dinv_ref[...]
    z_ref[0] = z[:, :C // 2]
    z_ref[1] = z[:, C // 2:]


def _pc(u, stats, g, b, gw, dinv):
    return pl.pallas_call(
        _pc_body,
        grid=(GRID,),
        in_specs=[
            _row_spec(C),
            _full_spec((2, C)),
            _full_spec((1, C)),
            _full_spec((1, C)),
            _full_spec((C, C)),
            _row_spec(1),
        ],
        out_specs=_z_spec(),
        out_shape=jax.ShapeDtypeStruct((NC, NP, C // 2), jnp.float32),
    )(u, stats, g, b, gw, dinv)


def _pd_body(t_ref, stats_ref, g_ref, b_ref, craw_ref, cstats_ref, pg_ref,
             pb_ref, mw_ref, mb_ref, u_ref, ustats_ref, acc):
    i = pl.program_id(0)
    h = jnp.maximum(_bn(t_ref[...], stats_ref[...], g_ref[0][None, :],
                        b_ref[0][None, :]), 0.0)
    cmu = cstats_ref[0] / N
    cvar = cstats_ref[1] / N - cmu * cmu
    cb = jnp.maximum((craw_ref[...] - cmu[None, :]) * lax.rsqrt(cvar + EPS)
                     * pg_ref[0][None, :] + pb_ref[0][None, :], 0.0)
    xc = jnp.concatenate([h, cb], axis=1)
    u = jnp.dot(xc, mw_ref[...], preferred_element_type=jnp.float32) \
        + mb_ref[0][None, :]
    u_ref[...] = u

    @pl.when(i == 0)
    def _():
        acc[...] = jnp.zeros_like(acc)

    um = jnp.where(_row_mask(i), u, 0.0)
    acc[0, :] += jnp.sum(um, axis=0)
    acc[1, :] += jnp.sum(um * u, axis=0)
    ustats_ref[...] = acc[...]


def _pd(t, stats, g, b, craw, cstats, pg, pbb, mw, mb):
    return pl.pallas_call(
        _pd_body,
        grid=(GRID,),
        in_specs=[
            _row_spec(C),
            _full_spec((2, C)),
            _full_spec((1, C)),
            _full_spec((1, C)),
            _row_spec(4),
            _full_spec((2, 4)),
            _full_spec((1, 4)),
            _full_spec((1, 4)),
            _full_spec((C + 4, C)),
            _full_spec((1, C)),
        ],
        out_specs=[_row_spec(C), _full_spec((2, C))],
        out_shape=[
            jax.ShapeDtypeStruct((NP, C), jnp.float32),
            jax.ShapeDtypeStruct((2, C), jnp.float32),
        ],
        scratch_shapes=[pltpu.VMEM((2, C), jnp.float32)],
    )(t, stats, g, b, craw, cstats, pg, pbb, mw, mb)


def _pe_body(u_ref, stats_ref, g_ref, b_ref, lw_ref, lb_ref, o_ref):
    y = _bn(u_ref[...], stats_ref[...], g_ref[0][None, :], b_ref[0][None, :])
    o_ref[...] = jnp.maximum(
        jnp.dot(y, lw_ref[...], preferred_element_type=jnp.float32)
        + lb_ref[0][None, :], 0.0)


def _pe(u, stats, g, b, lw, lb):
    return pl.pallas_call(
        _pe_body,
        grid=(GRID,),
        in_specs=[
            _row_spec(C),
            _full_spec((2, C)),
            _full_spec((1, C)),
            _full_spec((1, C)),
            _full_spec((C, 128)),
            _full_spec((1, 128)),
        ],
        out_specs=_row_spec(128),
        out_shape=jax.ShapeDtypeStruct((NP, 128), jnp.float32),
    )(u, stats, g, b, lw, lb)


def kernel(fx, fy, skip0, skip1, skip2, params, labels, edges_nn):
    f32 = jnp.float32
    feat = _fasm(skip0.reshape(B, C, PIX).astype(f32),
                 skip1.reshape(B, C, PIX).astype(f32),
                 skip2.reshape(B, C, PIX).astype(f32),
                 fx.reshape(B, 1, PIX).astype(f32),
                 fy.reshape(B, 1, PIX).astype(f32))
    lbl = labels.reshape(B, NS, 16, PW).astype(jnp.int32)

    src = jnp.pad(edges_nn[0].astype(jnp.int32), (0, EPAD - E))
    dst = jnp.pad(edges_nn[1].astype(jnp.int32), (0, EPAD - E))
    lab = jnp.pad(edges_nn[2].astype(jnp.int32), (0, EPAD - E),
                  constant_values=-1)

    p = params
    row = lambda a: a.reshape(1, -1).astype(f32)

    psum = _pool_kernel(feat, lbl).reshape(NC, NP, FWH)
    srcs4, tgt3 = _eprep(src, dst, lab)
    srcs4 = srcs4.reshape(NC, NS, NWIN, EWA)
    tgt3 = tgt3.reshape(NS, NWIN, EWA)
    degp = _deg_kernel(tgt3)
    craw, dinv, z0, cstats = _p1(psum, degp, p['gW0'].astype(f32))

    agg0 = _agg_kernel(z0.reshape(NC * NP, C // 2), srcs4, tgt3)
    t0, st0 = _pa(agg0, z0, dinv, row(p['gb0']))
    u1, su1 = _pb(t0, st0, row(p['gbn_g0']), row(p['gbn_b0']), psum,
                  p['mW0'].astype(f32), row(p['mb0']), 0)
    z1 = _pc(u1, su1, row(p['mbn_g0']), row(p['mbn_b0']),
             p['gW1'].astype(f32), dinv)

    agg1 = _agg_kernel(z1.reshape(NC * NP, C // 2), srcs4, tgt3)
    t1, st1 = _pa(agg1, z1, dinv, row(p['gb1']))
    u2, su2 = _pb(t1, st1, row(p['gbn_g1']), row(p['gbn_b1']), psum,
                  p['mW1'].astype(f32), row(p['mb1']), 1)
    z2 = _pc(u2, su2, row(p['mbn_g1']), row(p['mbn_b1']),
             p['gW2'].astype(f32), dinv)

    agg2 = _agg_kernel(z2.reshape(NC * NP, C // 2), srcs4, tgt3)
    t2, st2 = _pa(agg2, z2, dinv, row(p['gb2']))
    u3, su3 = _pd(t2, st2, row(p['gbn_g2']), row(p['gbn_b2']), craw, cstats,
                  row(p['pre_g']), row(p['pre_b']),
                  p['mW2'].astype(f32), row(p['mb2']))
    out = _pe(u3, su3, row(p['mbn_g2']), row(p['mbn_b2']),
              p['lW'].astype(f32), row(p['lb']))
    return out.reshape(B, SPAD, 128)[:, :S].reshape(N, 128)
